# Initial kernel scaffold; baseline (speedup 1.0000x reference)
import jax
import jax.numpy as jnp
from jax.experimental import pallas as pl


def kernel(nodes, all_neighbors, features, W1, W2, Wc):
    raise NotImplementedError("write your pallas kernel here")


# speedup vs baseline: 5.3946x; 5.3946x over previous
import functools

import jax
import jax.numpy as jnp
from jax import lax
from jax.experimental import pallas as pl
from jax.experimental.pallas import tpu as pltpu
from jax.experimental.pallas import tpu_sc as plsc

_NC = 2
_NS = 16
_NW = _NC * _NS
_LANES = 16

_S = 10

_SC_PARAMS = pltpu.CompilerParams(use_tc_tiling_on_sc=False)


def _widx():
    return lax.axis_index("s") * _NC + lax.axis_index("c")


def _sc_gather(table, idx, chunk=128):
    n_rows, d = table.shape
    b = idx.shape[0]
    per_w = b // _NW
    assert b % _NW == 0
    c = min(chunk, per_w)
    nchunk = per_w // c
    assert per_w % c == 0

    mesh = plsc.VectorSubcoreMesh(core_axis_name="c", subcore_axis_name="s")

    @functools.partial(
        pl.kernel,
        mesh=mesh,
        compiler_params=_SC_PARAMS,
        out_type=jax.ShapeDtypeStruct((b, d), table.dtype),
        scratch_types=[
            pltpu.VMEM((per_w,), jnp.int32),
            pltpu.VMEM((c, d), table.dtype),
            pltpu.SemaphoreType.DMA,
        ],
    )
    def k(table_hbm, idx_hbm, out_hbm, idx_v, buf, sem):
        base = _widx() * per_w
        pltpu.sync_copy(idx_hbm.at[pl.ds(base, per_w)], idx_v)

        def body(ci, carry):
            pltpu.async_copy(
                table_hbm.at[idx_v.at[pl.ds(ci * c, c)]], buf, sem
            ).wait()
            pltpu.sync_copy(buf, out_hbm.at[pl.ds(base + ci * c, c)])
            return carry

        lax.fori_loop(0, nchunk, body, 0)

    return k(table, idx)


def _sc_gather_sum(table, eidx, chunk=128):
    n_rows, d = table.shape
    s, b = eidx.shape
    per_w = b // _NW
    assert b % _NW == 0
    c = min(chunk, per_w)
    nchunk = per_w // c
    assert per_w % c == 0
    nv = d // _LANES

    mesh = plsc.VectorSubcoreMesh(core_axis_name="c", subcore_axis_name="s")

    @functools.partial(
        pl.kernel,
        mesh=mesh,
        compiler_params=_SC_PARAMS,
        out_type=jax.ShapeDtypeStruct((b, d), jnp.float32),
        scratch_types=[
            pltpu.VMEM((s, per_w), jnp.int32),
            pltpu.VMEM((c, d), jnp.float32),
            pltpu.VMEM((c, d), jnp.float32),
            pltpu.SemaphoreType.DMA,
        ],
    )
    def k(table_hbm, eidx_hbm, out_hbm, idx_v, buf, acc, sem):
        base = _widx() * per_w
        pltpu.sync_copy(eidx_hbm.at[:, pl.ds(base, per_w)], idx_v)

        def chunk_body(ci, carry):
            pltpu.async_copy(
                table_hbm.at[idx_v.at[0, pl.ds(ci * c, c)]], acc, sem
            ).wait()

            def j_body(j, cj):
                pltpu.async_copy(
                    table_hbm.at[idx_v.at[j, pl.ds(ci * c, c)]], buf, sem
                ).wait()

                def r_body(r, cr):
                    for v in range(nv):
                        sl = pl.ds(v * _LANES, _LANES)
                        plsc.addupdate(acc.at[r, sl], buf[r, sl])
                    return cr

                lax.fori_loop(0, c, r_body, 0)
                return cj

            lax.fori_loop(1, s, j_body, 0)
            pltpu.sync_copy(acc, out_hbm.at[pl.ds(base + ci * c, c)])
            return carry

        lax.fori_loop(0, nchunk, chunk_body, 0)

    return k(table, eidx)


def _tc_head(selff, nsum, w1a, w1b, w2a, w2b, wc, b_seeds):
    embed = w1a.shape[1]
    ncls = wc.shape[1]
    nb = 8
    bs = b_seeds // nb
    self_block0 = (b_seeds * _S) // bs

    def body(sn, nn, ss, ns, r1a, r1b, r2a, r2b, rc, o_ref):
        h1n = jnp.maximum(
            jnp.dot(sn[...], r1a[...], preferred_element_type=jnp.float32)
            + jnp.dot(nn[...], r1b[...], preferred_element_type=jnp.float32),
            0.0,
        )
        hsum = jnp.sum(h1n.reshape(bs, _S, embed), axis=1)
        h1s = jnp.maximum(
            jnp.dot(ss[...], r1a[...], preferred_element_type=jnp.float32)
            + jnp.dot(ns[...], r1b[...], preferred_element_type=jnp.float32),
            0.0,
        )
        emb = jnp.maximum(
            jnp.dot(h1s, r2a[...], preferred_element_type=jnp.float32)
            + jnp.dot(hsum, r2b[...], preferred_element_type=jnp.float32),
            0.0,
        )
        o_ref[...] = jax.nn.sigmoid(
            jnp.dot(emb, rc[...], preferred_element_type=jnp.float32)
        )

    nfeat = w1a.shape[0]
    wspec = lambda shp: pl.BlockSpec(shp, lambda i: (0, 0))
    return pl.pallas_call(
        body,
        grid=(nb,),
        in_specs=[
            pl.BlockSpec((bs * _S, nfeat), lambda i: (i, 0)),
            pl.BlockSpec((bs * _S, nfeat), lambda i: (i, 0)),
            pl.BlockSpec((bs, nfeat), lambda i: (i + self_block0, 0)),
            pl.BlockSpec((bs, nfeat), lambda i: (i + self_block0, 0)),
            wspec(w1a.shape),
            wspec(w1b.shape),
            wspec(w2a.shape),
            wspec(w2b.shape),
            wspec(wc.shape),
        ],
        out_specs=pl.BlockSpec((bs, ncls), lambda i: (i, 0)),
        out_shape=jax.ShapeDtypeStruct((b_seeds, ncls), jnp.float32),
    )(selff, nsum, selff, nsum, w1a, w1b, w2a, w2b, wc)


def kernel(nodes, all_neighbors, features, W1, W2, Wc):
    b = nodes.shape[0]
    nfeat = features.shape[1]
    embed = W1.shape[0]

    nodes = nodes.astype(jnp.int32)
    all_neighbors = all_neighbors.astype(jnp.int32)

    nb2 = _sc_gather(all_neighbors, nodes)
    ids_all = jnp.concatenate([nb2[:, :_S].reshape(-1), nodes])

    nbr1 = _sc_gather(all_neighbors, ids_all)
    eidx = nbr1[:, :_S].T

    selff = _sc_gather(features, ids_all)
    nsum = _sc_gather_sum(features, eidx)

    inv_s = jnp.float32(1.0 / _S)
    w1a = W1[:, :nfeat].T
    w1b = W1[:, nfeat:].T * inv_s
    w2a = W2[:, :embed].T
    w2b = W2[:, embed:].T * inv_s
    wct = Wc.T
    return _tc_head(selff, nsum, w1a, w1b, w2a, w2b, wct, b)


# speedup vs baseline: 6.4961x; 1.2042x over previous
import functools

import jax
import jax.numpy as jnp
from jax import lax
from jax.experimental import pallas as pl
from jax.experimental.pallas import tpu as pltpu
from jax.experimental.pallas import tpu_sc as plsc

_NC = 2
_NS = 16
_NW = _NC * _NS
_LANES = 16

_S = 10

_SC_PARAMS = pltpu.CompilerParams(use_tc_tiling_on_sc=False,
                                  needs_layout_passes=False)


def _widx():
    return lax.axis_index("s") * _NC + lax.axis_index("c")


def _sc_gather(table, idx, chunk=128):
    n_rows, d = table.shape
    b = idx.shape[0]
    per_w = b // _NW
    assert b % _NW == 0
    c = min(chunk, per_w)
    nchunk = per_w // c
    assert per_w % c == 0

    mesh = plsc.VectorSubcoreMesh(core_axis_name="c", subcore_axis_name="s")

    @functools.partial(
        pl.kernel,
        mesh=mesh,
        compiler_params=_SC_PARAMS,
        out_type=jax.ShapeDtypeStruct((b, d), table.dtype),
        scratch_types=[
            pltpu.VMEM((per_w,), jnp.int32),
            pltpu.VMEM((c, d), table.dtype),
            pltpu.SemaphoreType.DMA,
        ],
    )
    def k(table_hbm, idx_hbm, out_hbm, idx_v, buf, sem):
        base = _widx() * per_w
        pltpu.sync_copy(idx_hbm.at[pl.ds(base, per_w)], idx_v)

        def body(ci, carry):
            pltpu.async_copy(
                table_hbm.at[idx_v.at[pl.ds(ci * c, c)]], buf, sem
            ).wait()
            pltpu.sync_copy(buf, out_hbm.at[pl.ds(base + ci * c, c)])
            return carry

        lax.fori_loop(0, nchunk, body, 0)

    return k(table, idx)


def _sc_hop2(features, all_neighbors, ids, chunk=64):
    n_rows, d = features.shape
    deg = all_neighbors.shape[1]
    b = ids.shape[0]
    per_w = b // _NW
    assert b % _NW == 0
    c = chunk
    nch = per_w // c
    assert per_w % c == 0 and nch % 2 == 0
    nv = d // _LANES
    ng = c // _LANES

    mesh = plsc.VectorSubcoreMesh(core_axis_name="c", subcore_axis_name="s")

    @functools.partial(
        pl.kernel,
        mesh=mesh,
        compiler_params=_SC_PARAMS,
        out_type=jax.ShapeDtypeStruct((b, 2 * d), jnp.float32),
        scratch_types=[
            pltpu.VMEM((per_w,), jnp.int32),
            pltpu.VMEM((c, deg), jnp.int32),
            pltpu.VMEM((c, deg), jnp.int32),
            pltpu.VMEM((_S, c), jnp.int32),
            pltpu.VMEM((c, d), jnp.float32),
            pltpu.VMEM((c, d), jnp.float32),
            pltpu.VMEM((c, d), jnp.float32),
            pltpu.VMEM((c, d), jnp.float32),
            pltpu.VMEM((c, d), jnp.float32),
            pltpu.VMEM((c, d), jnp.float32),
            pltpu.SemaphoreType.DMA,
            pltpu.SemaphoreType.DMA,
            pltpu.SemaphoreType.DMA,
            pltpu.SemaphoreType.DMA,
            pltpu.SemaphoreType.DMA,
            pltpu.SemaphoreType.DMA,
            pltpu.SemaphoreType.DMA,
            pltpu.SemaphoreType.DMA,
            pltpu.SemaphoreType.DMA,
            pltpu.SemaphoreType.DMA,
            pltpu.SemaphoreType.DMA,
        ],
    )
    def k(feat_hbm, an_hbm, ids_hbm, out_hbm, ids_v, nbr0, nbr1, idx,
          selfb0, selfb1, acc0, acc1, jb0, jb1,
          sem_nbr0, sem_nbr1, sem_self0, sem_self1, sem_jA,
          sem_jb0, sem_jb1, sem_wS0, sem_wS1, sem_wA0, sem_wA1):
        nbr = (nbr0, nbr1)
        selfb = (selfb0, selfb1)
        acc = (acc0, acc1)
        jb = (jb0, jb1)
        sem_nbr = (sem_nbr0, sem_nbr1)
        sem_self = (sem_self0, sem_self1)
        sem_jb = (sem_jb0, sem_jb1)
        sem_wS = (sem_wS0, sem_wS1)
        sem_wA = (sem_wA0, sem_wA1)

        base = _widx() * per_w
        pltpu.sync_copy(ids_hbm.at[pl.ds(base, per_w)], ids_v)

        rows16 = jnp.arange(_LANES, dtype=jnp.int32)

        pltpu.async_copy(an_hbm.at[ids_v.at[pl.ds(0, c)]], nbr0, sem_nbr0)

        def do_chunk(t, p):
            ci = 2 * t + p

            @pl.when(t >= 1)
            def _():
                pltpu.make_async_copy(feat_hbm.at[pl.ds(0, c)], selfb[p],
                                      sem_wS[p]).wait()
                pltpu.make_async_copy(feat_hbm.at[pl.ds(0, c)], acc[p],
                                      sem_wA[p]).wait()

            pltpu.make_async_copy(an_hbm.at[ids_v.at[pl.ds(0, c)]],
                                  nbr[p], sem_nbr[p]).wait()

            @pl.when(ci + 1 < nch)
            def _():
                pltpu.async_copy(
                    an_hbm.at[ids_v.at[pl.ds((ci + 1) * c, c)]],
                    nbr[1 - p], sem_nbr[1 - p])

            for j in range(_S):
                cols = jnp.full((_LANES,), j, dtype=jnp.int32)
                for g in range(ng):
                    vals = plsc.load_gather(
                        nbr[p], [rows16 + (g * _LANES), cols])
                    idx[j, pl.ds(g * _LANES, _LANES)] = vals

            cp_acc = pltpu.async_copy(feat_hbm.at[idx.at[0]], acc[p], sem_jA)
            cp_self = pltpu.async_copy(
                feat_hbm.at[ids_v.at[pl.ds(ci * c, c)]], selfb[p], sem_self[p])
            jdesc = {1: pltpu.async_copy(feat_hbm.at[idx.at[1]], jb[0],
                                         sem_jb0)}
            cp_acc.wait()

            for j in range(1, _S):
                q = (j - 1) & 1
                if j + 1 < _S:
                    jdesc[j + 1] = pltpu.async_copy(
                        feat_hbm.at[idx.at[j + 1]], jb[1 - q], sem_jb[1 - q])
                jdesc[j].wait()

                def r_body(r, cr):
                    for v in range(nv):
                        sl = pl.ds(v * _LANES, _LANES)
                        plsc.addupdate(acc[p].at[r, sl], jb[q][r, sl])
                    return cr

                lax.fori_loop(0, c, r_body, 0)

            cp_self.wait()
            orow = base + ci * c
            pltpu.async_copy(selfb[p],
                             out_hbm.at[pl.ds(orow, c), pl.ds(0, d)],
                             sem_wS[p])
            pltpu.async_copy(acc[p],
                             out_hbm.at[pl.ds(orow, c), pl.ds(d, d)],
                             sem_wA[p])

        def pair(t, carry):
            do_chunk(t, 0)
            do_chunk(t, 1)
            return carry

        lax.fori_loop(0, nch // 2, pair, 0)

        for p in (0, 1):
            pltpu.make_async_copy(feat_hbm.at[pl.ds(0, c)], selfb[p],
                                  sem_wS[p]).wait()
            pltpu.make_async_copy(feat_hbm.at[pl.ds(0, c)], acc[p],
                                  sem_wA[p]).wait()

    return k(features, all_neighbors, ids)


def _tc_head(x, w1t, w2a, w2b, wc, b_seeds):
    twof = x.shape[1]
    embed = w1t.shape[1]
    ncls = wc.shape[1]
    nb = 8
    bs = b_seeds // nb
    self_block0 = (b_seeds * _S) // bs

    def body(xn, xs, r1, r2a, r2b, rc, o_ref):
        h1n = jnp.maximum(
            jnp.dot(xn[...], r1[...], preferred_element_type=jnp.float32), 0.0
        )
        hsum = jnp.sum(h1n.reshape(bs, _S, embed), axis=1)
        h1s = jnp.maximum(
            jnp.dot(xs[...], r1[...], preferred_element_type=jnp.float32), 0.0
        )
        emb = jnp.maximum(
            jnp.dot(h1s, r2a[...], preferred_element_type=jnp.float32)
            + jnp.dot(hsum, r2b[...], preferred_element_type=jnp.float32),
            0.0,
        )
        o_ref[...] = jax.nn.sigmoid(
            jnp.dot(emb, rc[...], preferred_element_type=jnp.float32)
        )

    wspec = lambda shp: pl.BlockSpec(shp, lambda i: (0, 0))
    return pl.pallas_call(
        body,
        grid=(nb,),
        in_specs=[
            pl.BlockSpec((bs * _S, twof), lambda i: (i, 0)),
            pl.BlockSpec((bs, twof), lambda i: (i + self_block0, 0)),
            wspec(w1t.shape),
            wspec(w2a.shape),
            wspec(w2b.shape),
            wspec(wc.shape),
        ],
        out_specs=pl.BlockSpec((bs, ncls), lambda i: (i, 0)),
        out_shape=jax.ShapeDtypeStruct((b_seeds, ncls), jnp.float32),
    )(x, x, w1t, w2a, w2b, wc)


def kernel(nodes, all_neighbors, features, W1, W2, Wc):
    b = nodes.shape[0]
    nfeat = features.shape[1]
    embed = W1.shape[0]

    nodes = nodes.astype(jnp.int32)
    all_neighbors = all_neighbors.astype(jnp.int32)

    nb2 = _sc_gather(all_neighbors, nodes)
    ids_all = jnp.concatenate([nb2[:, :_S].reshape(-1), nodes])

    x = _sc_hop2(features, all_neighbors, ids_all)

    inv_s = jnp.float32(1.0 / _S)
    w1t = jnp.concatenate([W1[:, :nfeat], W1[:, nfeat:] * inv_s], axis=1).T
    w2a = W2[:, :embed].T
    w2b = W2[:, embed:].T * inv_s
    wct = Wc.T
    return _tc_head(x, w1t, w2a, w2b, wct, b)


# speedup vs baseline: 7.2352x; 1.1138x over previous
import functools

import jax
import jax.numpy as jnp
from jax import lax
from jax.experimental import pallas as pl
from jax.experimental.pallas import tpu as pltpu
from jax.experimental.pallas import tpu_sc as plsc

_NC = 2
_NS = 16
_NW = _NC * _NS
_LANES = 16

_S = 10

_SC_PARAMS = pltpu.CompilerParams(use_tc_tiling_on_sc=False,
                                  needs_layout_passes=False)


def _widx():
    return lax.axis_index("s") * _NC + lax.axis_index("c")


def _sc_sage(features, all_neighbors, nodes):
    n_rows, d = features.shape
    deg = all_neighbors.shape[1]
    b = nodes.shape[0]
    sb = b // _NW
    c = 128
    per_w = sb * (_S + 1)
    nch = per_w // c
    assert per_w % c == 0 and sb % _LANES == 0 and sb * _S % c == 0
    nv = d // _LANES
    ng = c // _LANES
    nd = 3

    mesh = plsc.VectorSubcoreMesh(core_axis_name="c", subcore_axis_name="s")

    @functools.partial(
        pl.kernel,
        mesh=mesh,
        compiler_params=_SC_PARAMS,
        out_type=jax.ShapeDtypeStruct((b * (_S + 1), 2 * d), jnp.float32),
        scratch_types=[
            pltpu.VMEM((per_w,), jnp.int32),
            pltpu.VMEM((c, deg), jnp.int32),
            pltpu.VMEM((c, deg), jnp.int32),
            pltpu.VMEM((_S, c), jnp.int32),
            pltpu.VMEM((c, d), jnp.float32),
            pltpu.VMEM((c, d), jnp.float32),
            pltpu.VMEM((c, d), jnp.float32),
            pltpu.VMEM((c, d), jnp.float32),
            pltpu.VMEM((c, d), jnp.float32),
            pltpu.VMEM((c, d), jnp.float32),
            pltpu.VMEM((c, d), jnp.float32),
            pltpu.SemaphoreType.DMA,
            pltpu.SemaphoreType.DMA,
            pltpu.SemaphoreType.DMA,
            pltpu.SemaphoreType.DMA,
            pltpu.SemaphoreType.DMA,
            pltpu.SemaphoreType.DMA,
            pltpu.SemaphoreType.DMA,
            pltpu.SemaphoreType.DMA,
            pltpu.SemaphoreType.DMA,
            pltpu.SemaphoreType.DMA,
            pltpu.SemaphoreType.DMA,
        ],
    )
    def k(feat_hbm, an_hbm, nodes_hbm, out_hbm, ids_v, nbr0, nbr1, idx,
          selfb0, selfb1, acc0, acc1, jb0, jb1, jb2,
          sem_nbr0, sem_nbr1, sem_self0, sem_self1,
          sem_jb0, sem_jb1, sem_jb2, sem_wS0, sem_wS1, sem_wA0, sem_wA1):
        nbr = (nbr0, nbr1)
        selfb = (selfb0, selfb1)
        acc = (acc0, acc1)
        jb = (jb0, jb1, jb2)
        sem_nbr = (sem_nbr0, sem_nbr1)
        sem_self = (sem_self0, sem_self1)
        sem_jb = (sem_jb0, sem_jb1, sem_jb2)
        sem_wS = (sem_wS0, sem_wS1)
        sem_wA = (sem_wA0, sem_wA1)

        w = _widx()
        nbase = w * sb * _S
        sbase = b * _S + w * sb

        rows16 = jnp.arange(_LANES, dtype=jnp.int32)
        zero16 = jnp.zeros((_LANES,), dtype=jnp.float32)

        pltpu.sync_copy(nodes_hbm.at[pl.ds(w * sb, sb)],
                        ids_v.at[pl.ds(sb * _S, sb)])
        pltpu.async_copy(an_hbm.at[ids_v.at[pl.ds(sb * _S, sb)]],
                         nbr1, sem_nbr1)
        pltpu.make_async_copy(an_hbm.at[ids_v.at[pl.ds(sb * _S, sb)]],
                              nbr1, sem_nbr1).wait()
        for j in range(_S):
            cols = jnp.full((_LANES,), j, dtype=jnp.int32)
            for g in range(sb // _LANES):
                srows = rows16 + (g * _LANES)
                vals = plsc.load_gather(nbr1, [srows, cols])
                plsc.store_scatter(ids_v, [srows * _S + j], vals)

        pltpu.async_copy(an_hbm.at[ids_v.at[pl.ds(0, c)]], nbr0, sem_nbr0)

        def do_chunk(t, p):
            ci = 2 * t + p

            @pl.when(t >= 1)
            def _():
                pltpu.make_async_copy(feat_hbm.at[pl.ds(0, c)], selfb[p],
                                      sem_wS[p]).wait()
                pltpu.make_async_copy(feat_hbm.at[pl.ds(0, c)], acc[p],
                                      sem_wA[p]).wait()

            pltpu.make_async_copy(an_hbm.at[ids_v.at[pl.ds(0, c)]],
                                  nbr[p], sem_nbr[p]).wait()

            @pl.when(ci + 1 < nch)
            def _():
                pltpu.async_copy(
                    an_hbm.at[ids_v.at[pl.ds((ci + 1) * c, c)]],
                    nbr[1 - p], sem_nbr[1 - p])

            for j in range(_S):
                cols = jnp.full((_LANES,), j, dtype=jnp.int32)
                for g in range(ng):
                    vals = plsc.load_gather(
                        nbr[p], [rows16 + (g * _LANES), cols])
                    idx[j, pl.ds(g * _LANES, _LANES)] = vals

            jdesc = {}
            for j in range(nd):
                jdesc[j] = pltpu.async_copy(feat_hbm.at[idx.at[j]],
                                            jb[j % nd], sem_jb[j % nd])
            cp_self = pltpu.async_copy(
                feat_hbm.at[ids_v.at[pl.ds(ci * c, c)]], selfb[p],
                sem_self[p])

            def z_body(r, cz):
                for v in range(nv):
                    acc[p][r, pl.ds(v * _LANES, _LANES)] = zero16
                return cz

            lax.fori_loop(0, c, z_body, 0)

            for j in range(_S):
                q = j % nd
                jdesc[j].wait()

                def r_body(r, cr):
                    for v in range(nv):
                        sl = pl.ds(v * _LANES, _LANES)
                        plsc.addupdate(acc[p].at[r, sl], jb[q][r, sl])
                    return cr

                lax.fori_loop(0, c, r_body, 0)
                if j + nd < _S:
                    jdesc[j + nd] = pltpu.async_copy(
                        feat_hbm.at[idx.at[j + nd]], jb[q], sem_jb[q])

            cp_self.wait()
            orow = jnp.where(ci < nch - 1, nbase + ci * c, sbase)
            pltpu.async_copy(selfb[p],
                             out_hbm.at[pl.ds(orow, c), pl.ds(0, d)],
                             sem_wS[p])
            pltpu.async_copy(acc[p],
                             out_hbm.at[pl.ds(orow, c), pl.ds(d, d)],
                             sem_wA[p])

        def pair(t, carry):
            do_chunk(t, 0)
            do_chunk(t, 1)
            return carry

        lax.fori_loop(0, nch // 2, pair, 0)
        if nch % 2:
            do_chunk(nch // 2, 0)

        for p in (0, 1):
            pltpu.make_async_copy(feat_hbm.at[pl.ds(0, c)], selfb[p],
                                  sem_wS[p]).wait()
            pltpu.make_async_copy(feat_hbm.at[pl.ds(0, c)], acc[p],
                                  sem_wA[p]).wait()

    return k(features, all_neighbors, nodes)


def _tc_head(x, w1t, w2a, w2b, wc, b_seeds):
    twof = x.shape[1]
    embed = w1t.shape[1]
    ncls = wc.shape[1]
    nb = 8
    bs = b_seeds // nb
    self_block0 = (b_seeds * _S) // bs

    def body(xn, xs, r1, r2a, r2b, rc, o_ref):
        h1n = jnp.maximum(
            jnp.dot(xn[...], r1[...], preferred_element_type=jnp.float32), 0.0
        )
        hsum = jnp.sum(h1n.reshape(bs, _S, embed), axis=1)
        h1s = jnp.maximum(
            jnp.dot(xs[...], r1[...], preferred_element_type=jnp.float32), 0.0
        )
        emb = jnp.maximum(
            jnp.dot(h1s, r2a[...], preferred_element_type=jnp.float32)
            + jnp.dot(hsum, r2b[...], preferred_element_type=jnp.float32),
            0.0,
        )
        o_ref[...] = jax.nn.sigmoid(
            jnp.dot(emb, rc[...], preferred_element_type=jnp.float32)
        )

    wspec = lambda shp: pl.BlockSpec(shp, lambda i: (0, 0))
    return pl.pallas_call(
        body,
        grid=(nb,),
        in_specs=[
            pl.BlockSpec((bs * _S, twof), lambda i: (i, 0)),
            pl.BlockSpec((bs, twof), lambda i: (i + self_block0, 0)),
            wspec(w1t.shape),
            wspec(w2a.shape),
            wspec(w2b.shape),
            wspec(wc.shape),
        ],
        out_specs=pl.BlockSpec((bs, ncls), lambda i: (i, 0)),
        out_shape=jax.ShapeDtypeStruct((b_seeds, ncls), jnp.float32),
    )(x, x, w1t, w2a, w2b, wc)


def kernel(nodes, all_neighbors, features, W1, W2, Wc):
    b = nodes.shape[0]
    nfeat = features.shape[1]
    embed = W1.shape[0]

    nodes = nodes.astype(jnp.int32)
    all_neighbors = all_neighbors.astype(jnp.int32)

    x = _sc_sage(features, all_neighbors, nodes)

    inv_s = jnp.float32(1.0 / _S)
    w1t = jnp.concatenate([W1[:, :nfeat], W1[:, nfeat:] * inv_s], axis=1).T
    w2a = W2[:, :embed].T
    w2b = W2[:, embed:].T * inv_s
    wct = Wc.T
    return _tc_head(x, w1t, w2a, w2b, wct, b)


# speedup vs baseline: 8.8238x; 1.2196x over previous
import functools

import jax
import jax.numpy as jnp
from jax import lax
from jax.experimental import pallas as pl
from jax.experimental.pallas import tpu as pltpu
from jax.experimental.pallas import tpu_sc as plsc

_NC = 2
_NS = 16
_NW = _NC * _NS
_LANES = 16

_S = 10

_SC_PARAMS = pltpu.CompilerParams(use_tc_tiling_on_sc=False,
                                  needs_layout_passes=False)


def _widx():
    return lax.axis_index("s") * _NC + lax.axis_index("c")


def _sc_sage(features, all_neighbors, nodes):
    n_rows, d = features.shape
    deg = all_neighbors.shape[1]
    b = nodes.shape[0]
    sb = b // _NW
    c = 128
    per_w = sb * (_S + 1)
    nch = per_w // c
    assert per_w % c == 0 and sb % _LANES == 0 and sb * _S % c == 0
    nv = d // _LANES
    ng = c // _LANES
    nd = 3

    mesh = plsc.VectorSubcoreMesh(core_axis_name="c", subcore_axis_name="s")

    @functools.partial(
        pl.kernel,
        mesh=mesh,
        compiler_params=_SC_PARAMS,
        out_type=(jax.ShapeDtypeStruct((b * (_S + 1), d), jnp.float32),
                  jax.ShapeDtypeStruct((b * (_S + 1), d), jnp.float32)),
        scratch_types=[
            pltpu.VMEM((per_w,), jnp.int32),
            pltpu.VMEM((c, deg), jnp.int32),
            pltpu.VMEM((c, deg), jnp.int32),
            pltpu.VMEM((_S, c), jnp.int32),
            pltpu.VMEM((c, d), jnp.float32),
            pltpu.VMEM((c, d), jnp.float32),
            pltpu.VMEM((c, d), jnp.float32),
            pltpu.VMEM((c, d), jnp.float32),
            pltpu.VMEM((c, d), jnp.float32),
            pltpu.VMEM((c, d), jnp.float32),
            pltpu.VMEM((c, d), jnp.float32),
            pltpu.SemaphoreType.DMA,
            pltpu.SemaphoreType.DMA,
            pltpu.SemaphoreType.DMA,
            pltpu.SemaphoreType.DMA,
            pltpu.SemaphoreType.DMA,
            pltpu.SemaphoreType.DMA,
            pltpu.SemaphoreType.DMA,
            pltpu.SemaphoreType.DMA,
            pltpu.SemaphoreType.DMA,
            pltpu.SemaphoreType.DMA,
            pltpu.SemaphoreType.DMA,
            pltpu.SemaphoreType.DMA,
        ],
    )
    def k(feat_hbm, an_hbm, nodes_hbm, self_hbm, nsum_hbm, ids_v, nbr0, nbr1, idx,
          selfb0, selfb1, acc0, acc1, jb0, jb1, jb2,
          sem_nbr0, sem_nbr1, sem_self0, sem_self1, sem_jA,
          sem_jb0, sem_jb1, sem_jb2, sem_wS0, sem_wS1, sem_wA0, sem_wA1):
        nbr = (nbr0, nbr1)
        selfb = (selfb0, selfb1)
        acc = (acc0, acc1)
        jb = (jb0, jb1, jb2)
        sem_nbr = (sem_nbr0, sem_nbr1)
        sem_self = (sem_self0, sem_self1)
        sem_jb = (sem_jb0, sem_jb1, sem_jb2)
        sem_wS = (sem_wS0, sem_wS1)
        sem_wA = (sem_wA0, sem_wA1)

        w = _widx()
        nbase = w * sb * _S
        sbase = b * _S + w * sb

        rows16 = jnp.arange(_LANES, dtype=jnp.int32)

        pltpu.sync_copy(nodes_hbm.at[pl.ds(w * sb, sb)],
                        ids_v.at[pl.ds(sb * _S, sb)])
        pltpu.async_copy(an_hbm.at[ids_v.at[pl.ds(sb * _S, sb)]],
                         nbr1, sem_nbr1)
        pltpu.make_async_copy(an_hbm.at[ids_v.at[pl.ds(sb * _S, sb)]],
                              nbr1, sem_nbr1).wait()
        for j in range(_S):
            cols = jnp.full((_LANES,), j, dtype=jnp.int32)
            for g in range(sb // _LANES):
                srows = rows16 + (g * _LANES)
                vals = plsc.load_gather(nbr1, [srows, cols])
                plsc.store_scatter(ids_v, [srows * _S + j], vals)

        pltpu.async_copy(an_hbm.at[ids_v.at[pl.ds(0, c)]], nbr0, sem_nbr0)

        def do_chunk(t, p):
            ci = 2 * t + p

            @pl.when(t >= 1)
            def _():
                pltpu.make_async_copy(feat_hbm.at[pl.ds(0, c)], selfb[p],
                                      sem_wS[p]).wait()
                pltpu.make_async_copy(feat_hbm.at[pl.ds(0, c)], acc[p],
                                      sem_wA[p]).wait()

            pltpu.make_async_copy(an_hbm.at[ids_v.at[pl.ds(0, c)]],
                                  nbr[p], sem_nbr[p]).wait()

            @pl.when(ci + 1 < nch)
            def _():
                pltpu.async_copy(
                    an_hbm.at[ids_v.at[pl.ds((ci + 1) * c, c)]],
                    nbr[1 - p], sem_nbr[1 - p])

            for j in range(_S):
                cols = jnp.full((_LANES,), j, dtype=jnp.int32)
                for g in range(ng):
                    vals = plsc.load_gather(
                        nbr[p], [rows16 + (g * _LANES), cols])
                    idx[j, pl.ds(g * _LANES, _LANES)] = vals

            cp_acc = pltpu.async_copy(feat_hbm.at[idx.at[0]], acc[p], sem_jA)
            jdesc = {}
            for j in range(1, 1 + nd):
                jdesc[j] = pltpu.async_copy(feat_hbm.at[idx.at[j]],
                                            jb[(j - 1) % nd],
                                            sem_jb[(j - 1) % nd])
            cp_self = pltpu.async_copy(
                feat_hbm.at[ids_v.at[pl.ds(ci * c, c)]], selfb[p],
                sem_self[p])
            cp_acc.wait()

            for j in range(1, _S):
                q = (j - 1) % nd
                jdesc[j].wait()

                def r_body(r, cr):
                    for v in range(nv):
                        sl = pl.ds(v * _LANES, _LANES)
                        plsc.addupdate(acc[p].at[r, sl], jb[q][r, sl])
                    return cr

                lax.fori_loop(0, c, r_body, 0)
                if j + nd < _S:
                    jdesc[j + nd] = pltpu.async_copy(
                        feat_hbm.at[idx.at[j + nd]], jb[q], sem_jb[q])

            cp_self.wait()
            orow = jnp.where(ci < nch - 1, nbase + ci * c, sbase)
            pltpu.async_copy(selfb[p], self_hbm.at[pl.ds(orow, c)], sem_wS[p])
            pltpu.async_copy(acc[p], nsum_hbm.at[pl.ds(orow, c)], sem_wA[p])

        def pair(t, carry):
            do_chunk(t, 0)
            do_chunk(t, 1)
            return carry

        lax.fori_loop(0, nch // 2, pair, 0)
        if nch % 2:
            do_chunk(nch // 2, 0)

        for p in (0, 1):
            pltpu.make_async_copy(feat_hbm.at[pl.ds(0, c)], selfb[p],
                                  sem_wS[p]).wait()
            pltpu.make_async_copy(feat_hbm.at[pl.ds(0, c)], acc[p],
                                  sem_wA[p]).wait()

    return k(features, all_neighbors, nodes)


def _tc_head(selff, nsum, w1a, w1b, w2a, w2b, wc, b_seeds):
    nfeat = selff.shape[1]
    embed = w1a.shape[1]
    ncls = wc.shape[1]
    nb = 8
    bs = b_seeds // nb
    self_block0 = (b_seeds * _S) // bs

    def body(sn, nn, ss, ns, r1a, r1b, r2a, r2b, rc, o_ref):
        h1n = jnp.maximum(
            jnp.dot(sn[...], r1a[...], preferred_element_type=jnp.float32)
            + jnp.dot(nn[...], r1b[...], preferred_element_type=jnp.float32),
            0.0,
        )
        hsum = jnp.sum(h1n.reshape(bs, _S, embed), axis=1)
        h1s = jnp.maximum(
            jnp.dot(ss[...], r1a[...], preferred_element_type=jnp.float32)
            + jnp.dot(ns[...], r1b[...], preferred_element_type=jnp.float32),
            0.0,
        )
        emb = jnp.maximum(
            jnp.dot(h1s, r2a[...], preferred_element_type=jnp.float32)
            + jnp.dot(hsum, r2b[...], preferred_element_type=jnp.float32),
            0.0,
        )
        o_ref[...] = jax.nn.sigmoid(
            jnp.dot(emb, rc[...], preferred_element_type=jnp.float32)
        )

    wspec = lambda shp: pl.BlockSpec(shp, lambda i: (0, 0))
    return pl.pallas_call(
        body,
        grid=(nb,),
        in_specs=[
            pl.BlockSpec((bs * _S, nfeat), lambda i: (i, 0)),
            pl.BlockSpec((bs * _S, nfeat), lambda i: (i, 0)),
            pl.BlockSpec((bs, nfeat), lambda i: (i + self_block0, 0)),
            pl.BlockSpec((bs, nfeat), lambda i: (i + self_block0, 0)),
            wspec(w1a.shape),
            wspec(w1b.shape),
            wspec(w2a.shape),
            wspec(w2b.shape),
            wspec(wc.shape),
        ],
        out_specs=pl.BlockSpec((bs, ncls), lambda i: (i, 0)),
        out_shape=jax.ShapeDtypeStruct((b_seeds, ncls), jnp.float32),
    )(selff, nsum, selff, nsum, w1a, w1b, w2a, w2b, wc)


def kernel(nodes, all_neighbors, features, W1, W2, Wc):
    b = nodes.shape[0]
    nfeat = features.shape[1]
    embed = W1.shape[0]

    nodes = nodes.astype(jnp.int32)
    all_neighbors = all_neighbors.astype(jnp.int32)

    selff, nsum = _sc_sage(features, all_neighbors, nodes)

    inv_s = jnp.float32(1.0 / _S)
    w1a = W1[:, :nfeat].T
    w1b = W1[:, nfeat:].T * inv_s
    w2a = W2[:, :embed].T
    w2b = W2[:, embed:].T * inv_s
    wct = Wc.T
    return _tc_head(selff, nsum, w1a, w1b, w2a, w2b, wct, b)
